# baseline (speedup 1.0000x reference)
import functools

import jax
import jax.numpy as jnp
from jax import lax
from jax.experimental import pallas as pl
from jax.experimental.pallas import tpu as pltpu
from jax.experimental.pallas import tpu_sc as plsc

N = 1024
NLANES = 16
NCORES = 2
NSUBCORES = 16
NW = NCORES * NSUBCORES
SPW = N // NW
CSPLIT = 2
NBUF = 3
NOBUF = 2


def _make_sc_kernel(NIMG, C, H, W, NSAMP=N):
    SPW_ = NSAMP // NW
    CH = C // CSPLIT
    NT = SPW_ * CSPLIT
    mesh = plsc.VectorSubcoreMesh(core_axis_name="c", subcore_axis_name="s")

    OHS = C * W // 128

    @functools.partial(
        pl.kernel,
        mesh=mesh,
        out_type=jax.ShapeDtypeStruct((NSAMP, C * W // 128, 128),
                                      jnp.float32),
        compiler_params=pltpu.CompilerParams(use_tc_tiling_on_sc=True),
        scratch_types=[
            pltpu.VMEM((SPW_, NLANES), jnp.int32),
            pltpu.VMEM((SPW_, 2, NLANES), jnp.float32),
            pltpu.VMEM((NBUF, 2, CH, W), jnp.float32),
            pltpu.VMEM((NOBUF, OHS, 128), jnp.float32),
        ]
        + [pltpu.SemaphoreType.DMA for _ in range(NBUF + NOBUF)],
    )
    def sc_kernel(x_hbm, ih_hbm, wts_hbm, out_hbm,
                  ih_v, w_v, taps_v, obuf_v,
                  g0, g1, g2, o0, o1):
        wid = lax.axis_index("s") * NCORES + lax.axis_index("c")
        base = wid * SPW_
        pltpu.sync_copy(ih_hbm.at[pl.ds(base, SPW_)], ih_v)
        pltpu.sync_copy(wts_hbm.at[pl.ds(base, SPW_)], w_v)

        gsems = (g0, g1, g2)
        osems = (o0, o1)

        def tap_srcs(t):
            s = t // CSPLIT
            ch = t % CSPLIT
            v = ih_v[s]
            c0 = ch * CH
            return (x_hbm.at[v[0], pl.ds(c0, CH), v[1], :],
                    x_hbm.at[v[2], pl.ds(c0, CH), v[3], :])

        def start_gather(t, par, sem):
            a, b = tap_srcs(t)
            pltpu.async_copy(a, taps_v.at[par, 0], sem)
            pltpu.async_copy(b, taps_v.at[par, 1], sem)

        def wait_gather(t, par, sem):
            a, b = tap_srcs(t)
            pltpu.make_async_copy(a, taps_v.at[par, 0], sem).wait()
            pltpu.make_async_copy(b, taps_v.at[par, 1], sem).wait()

        def for_par(parval, fn):
            for i in range(NBUF):
                @pl.when(parval == i)
                def _(i=i):
                    fn(i, gsems[i])

        def for_opar(oparval, fn):
            for i in range(NOBUF):
                @pl.when(oparval == i)
                def _(i=i):
                    fn(i, osems[i])

        start_gather(0, 0, gsems[0])
        start_gather(1, 1, gsems[1])

        def unit(t, _):
            s = t // CSPLIT
            ch = t % CSPLIT
            par = t % NBUF
            opar = s % NOBUF

            for_par(par, lambda i, sem: wait_gather(t, i, sem))

            @pl.when(t + 2 < NT)
            def _():
                for_par((t + 2) % NBUF,
                        lambda i, sem: start_gather(t + 2, i, sem))

            @pl.when((ch == 0) & (s >= NOBUF))
            def _():
                for_opar(opar, lambda i, sem: pltpu.make_async_copy(
                    obuf_v.at[i], out_hbm.at[base + s - NOBUF], sem).wait())

            w0 = w_v[s, 0]
            w1 = w_v[s, 1]
            qbase = ch * (OHS // CSPLIT)

            @plsc.parallel_loop(0, CH // 4, unroll=2)
            def sblock(S):
                r0 = 4 * S
                q0 = qbase + 7 * S
                for k in range(56):
                    a = taps_v[par, 0, r0 + k // 14,
                               pl.ds((k % 14) * NLANES, NLANES)]
                    b = taps_v[par, 1, r0 + k // 14,
                               pl.ds((k % 14) * NLANES, NLANES)]
                    obuf_v[opar, q0 + k // 8,
                           pl.ds((k % 8) * NLANES, NLANES)] = (
                        w0 * a + w1 * b)

            @pl.when(ch == CSPLIT - 1)
            def _():
                for_opar(opar, lambda i, sem: pltpu.async_copy(
                    obuf_v.at[i], out_hbm.at[base + s], sem))

            return 0

        lax.fori_loop(0, NT, unit, 0)

        for s in (SPW_ - 2, SPW_ - 1):
            pltpu.make_async_copy(
                obuf_v.at[s % NOBUF], out_hbm.at[base + s],
                osems[s % NOBUF]).wait()

    return sc_kernel


def kernel(x, image_num, image_ids, rows):
    del image_num, image_ids
    NIMG, C, H, W = x.shape
    NROW = NIMG * H

    iy = rows - 0.5
    iy0 = jnp.floor(iy)
    w1 = iy - iy0
    w0 = 1.0 - w1
    y0 = iy0.astype(jnp.int32)
    y1 = y0 + 1
    w0 = jnp.where((y0 >= 0) & (y0 <= NROW - 1), 0.5 * w0, 0.0)
    w1 = jnp.where((y1 >= 0) & (y1 <= NROW - 1), 0.5 * w1, 0.0)
    y0c = jnp.clip(y0, 0, NROW - 1)
    y1c = jnp.clip(y1, 0, NROW - 1)

    ih = jnp.stack(
        [y0c // H, y0c % H, y1c // H, y1c % H], axis=-1)
    ih = jnp.pad(ih, ((0, 0), (0, NLANES - 4)))
    wts = jnp.broadcast_to(
        jnp.stack([w0, w1], axis=1)[:, :, None], (N, 2, NLANES))

    out = _make_sc_kernel(NIMG, C, H, W)(x, ih, wts)
    return jnp.reshape(out, (N, C, W))


# speedup vs baseline: 1.2062x; 1.2062x over previous
import functools

import jax
import jax.numpy as jnp
from jax import lax
from jax.experimental import pallas as pl
from jax.experimental.pallas import tpu as pltpu
from jax.experimental.pallas import tpu_sc as plsc

N = 1024
NLANES = 16
NCORES = 2
NSUBCORES = 16
NW = NCORES * NSUBCORES
SPW = N // NW
CSPLIT = 2
NBUF = 3
NOBUF = 2


def _make_sc_kernel(NIMG, C, H, W, NSAMP=N):
    SPW_ = NSAMP // NW
    CH = C // CSPLIT
    NT = SPW_ * CSPLIT
    mesh = plsc.VectorSubcoreMesh(core_axis_name="c", subcore_axis_name="s")

    OHS = C * W // 128

    @functools.partial(
        pl.kernel,
        mesh=mesh,
        out_type=jax.ShapeDtypeStruct((NSAMP, C * W // 128, 128),
                                      jnp.float32),
        compiler_params=pltpu.CompilerParams(use_tc_tiling_on_sc=True),
        scratch_types=[
            pltpu.VMEM((SPW_, NLANES), jnp.int32),
            pltpu.VMEM((SPW_, 2, NLANES), jnp.float32),
            pltpu.VMEM((NBUF, 2, CH, W), jnp.float32),
            pltpu.VMEM((NOBUF, OHS, 128), jnp.float32),
        ]
        + [pltpu.SemaphoreType.DMA for _ in range(NBUF + NOBUF)],
    )
    def sc_kernel(x_hbm, ih_hbm, wts_hbm, out_hbm,
                  ih_v, w_v, taps_v, obuf_v,
                  g0, g1, g2, o0, o1):
        wid = lax.axis_index("s") * NCORES + lax.axis_index("c")
        base = wid * SPW_
        pltpu.sync_copy(ih_hbm.at[pl.ds(base, SPW_)], ih_v)
        pltpu.sync_copy(wts_hbm.at[pl.ds(base, SPW_)], w_v)

        gsems = (g0, g1, g2)
        osems = (o0, o1)

        def tap_srcs(t):
            s = t // CSPLIT
            ch = t % CSPLIT
            v = ih_v[s]
            c0 = ch * CH
            return (x_hbm.at[v[0], pl.ds(c0, CH), v[1], :],
                    x_hbm.at[v[2], pl.ds(c0, CH), v[3], :])

        def start_gather(t, par, sem):
            a, b = tap_srcs(t)
            pltpu.async_copy(a, taps_v.at[par, 0], sem)
            pltpu.async_copy(b, taps_v.at[par, 1], sem)

        def wait_gather(t, par, sem):
            a, b = tap_srcs(t)
            pltpu.make_async_copy(a, taps_v.at[par, 0], sem).wait()
            pltpu.make_async_copy(b, taps_v.at[par, 1], sem).wait()

        def for_par(parval, fn):
            for i in range(NBUF):
                @pl.when(parval == i)
                def _(i=i):
                    fn(i, gsems[i])

        def for_opar(oparval, fn):
            for i in range(NOBUF):
                @pl.when(oparval == i)
                def _(i=i):
                    fn(i, osems[i])

        start_gather(0, 0, gsems[0])
        start_gather(1, 1, gsems[1])

        def unit(t, _):
            s = t // CSPLIT
            ch = t % CSPLIT
            par = t % NBUF
            opar = s % NOBUF

            for_par(par, lambda i, sem: wait_gather(t, i, sem))

            @pl.when(t + 2 < NT)
            def _():
                for_par((t + 2) % NBUF,
                        lambda i, sem: start_gather(t + 2, i, sem))

            @pl.when((ch == 0) & (s >= NOBUF))
            def _():
                for_opar(opar, lambda i, sem: pltpu.make_async_copy(
                    obuf_v.at[i], out_hbm.at[base + s - NOBUF], sem).wait())

            w0 = w_v[s, 0]
            w1 = w_v[s, 1]
            qbase = ch * (OHS // CSPLIT)

            @plsc.parallel_loop(0, CH // 4, unroll=1)
            def sblock(S):
                r0 = 4 * S
                q0 = qbase + 7 * S
                for k in range(56):
                    a = taps_v[par, 0, r0 + k // 14,
                               pl.ds((k % 14) * NLANES, NLANES)]
                    b = taps_v[par, 1, r0 + k // 14,
                               pl.ds((k % 14) * NLANES, NLANES)]
                    obuf_v[opar, q0 + k // 8,
                           pl.ds((k % 8) * NLANES, NLANES)] = (
                        w0 * a + w1 * b)

            @pl.when(ch == CSPLIT - 1)
            def _():
                for_opar(opar, lambda i, sem: pltpu.async_copy(
                    obuf_v.at[i], out_hbm.at[base + s], sem))

            return 0

        lax.fori_loop(0, NT, unit, 0)

        for s in (SPW_ - 2, SPW_ - 1):
            pltpu.make_async_copy(
                obuf_v.at[s % NOBUF], out_hbm.at[base + s],
                osems[s % NOBUF]).wait()

    return sc_kernel


def kernel(x, image_num, image_ids, rows):
    del image_num, image_ids
    NIMG, C, H, W = x.shape
    NROW = NIMG * H

    iy = rows - 0.5
    iy0 = jnp.floor(iy)
    w1 = iy - iy0
    w0 = 1.0 - w1
    y0 = iy0.astype(jnp.int32)
    y1 = y0 + 1
    w0 = jnp.where((y0 >= 0) & (y0 <= NROW - 1), 0.5 * w0, 0.0)
    w1 = jnp.where((y1 >= 0) & (y1 <= NROW - 1), 0.5 * w1, 0.0)
    y0c = jnp.clip(y0, 0, NROW - 1)
    y1c = jnp.clip(y1, 0, NROW - 1)

    ih = jnp.stack(
        [y0c // H, y0c % H, y1c // H, y1c % H], axis=-1)
    ih = jnp.pad(ih, ((0, 0), (0, NLANES - 4)))
    wts = jnp.broadcast_to(
        jnp.stack([w0, w1], axis=1)[:, :, None], (N, 2, NLANES))

    out = _make_sc_kernel(NIMG, C, H, W)(x, ih, wts)
    return jnp.reshape(out, (N, C, W))
